# Initial kernel scaffold; baseline (speedup 1.0000x reference)
import jax
import jax.numpy as jnp
from jax.experimental import pallas as pl


def kernel(params, params_default):
    raise NotImplementedError("write your pallas kernel here")


# speedup vs baseline: 3.5209x; 3.5209x over previous
import functools

import jax
import jax.numpy as jnp
from jax.experimental import pallas as pl

_B = 262144
_F = 32
_T = 64
_R = 2048


def _body(d_ref, p_ref, o_ref):
    tail = jnp.broadcast_to(d_ref[0:1, _F:_T], (_R, _T - _F))
    o_ref[...] = jnp.concatenate([p_ref[...], tail], axis=1)


@jax.jit
def kernel(params, params_default):
    d2 = params_default.reshape(1, _T)
    grid = _B // _R
    out = pl.pallas_call(
        _body,
        grid=(grid,),
        in_specs=[
            pl.BlockSpec((1, _T), lambda i: (0, 0)),
            pl.BlockSpec((_R, _F), lambda i: (i, 0)),
        ],
        out_specs=pl.BlockSpec((_R, _T), lambda i: (i, 0)),
        out_shape=jax.ShapeDtypeStruct((_B, _T), jnp.float32),
    )(d2, params)
    return out
